# baseline (speedup 1.0000x reference)
import functools

import jax
import jax.numpy as jnp
from jax import lax
from jax.experimental import pallas as pl
from jax.experimental.pallas import tpu as pltpu
from jax.experimental.pallas import tpu_sc as plsc

N_TABLES = 26
DIM = 32
NC, NS = 2, 16
NW = NC * NS
SUB = 128
SUBG = SUB // 16
QS = 4


def _gather_body(pidx_hbm, sub_hbm, packed_hbm, out_hbm, pidx_v, sub_v, rows_v,
                 stage_v, sem):
    wid = lax.axis_index("s") * NC + lax.axis_index("c")
    pltpu.sync_copy(pidx_hbm.at[:, wid], pidx_v)
    pltpu.sync_copy(sub_hbm.at[:, wid], sub_v)

    lane = lax.iota(jnp.int32, 16)
    prow_base = lane >> 2
    pcol_base = (lane & 3) * DIM

    @pl.loop(0, N_TABLES * QS)
    def _task(t):
        j = t // QS
        q = lax.rem(t, QS)
        pltpu.async_copy(packed_hbm.at[pidx_v.at[j, q]], rows_v, sem).wait()

        @pl.loop(0, SUBG)
        def _group(g):
            row_idx = g * 16 + lane
            prow = g * 4 + prow_base
            sub_off = sub_v[j, q, pl.ds(g * 16, 16)]
            for d in range(DIM):
                val = plsc.load_gather(rows_v, [row_idx, sub_off + d])
                plsc.store_scatter(stage_v, [prow, pcol_base + d], val)

        pltpu.sync_copy(
            stage_v, out_hbm.at[j, pl.ds(wid * 128 + q * 32, 32), :]
        )


def _sc_gather(pidx, sub_off, packed):
    b = NW * QS * SUB
    mesh = plsc.VectorSubcoreMesh(core_axis_name="c", subcore_axis_name="s")
    return pl.kernel(
        _gather_body,
        out_type=jax.ShapeDtypeStruct((N_TABLES, b // 4, 128), jnp.float32),
        mesh=mesh,
        scratch_types=[
            pltpu.VMEM((N_TABLES, QS, SUB), jnp.int32),
            pltpu.VMEM((N_TABLES, QS, SUB), jnp.int32),
            pltpu.VMEM((SUB, 128), jnp.float32),
            pltpu.VMEM((32, 128), jnp.float32),
            pltpu.SemaphoreType.DMA,
        ],
        compiler_params=pltpu.CompilerParams(
            use_tc_tiling_on_sc=False, needs_layout_passes=False
        ),
    )(pidx, sub_off, packed)


def _mm_body(emb_ref, w_ref, b_ref, out_ref):
    acc = b_ref[...]
    for j in range(N_TABLES):
        acc = acc + jnp.dot(
            emb_ref[j], w_ref[j], preferred_element_type=jnp.float32
        )
    out_ref[...] = acc


def _tc_matmul(emb, wbig, b2d):
    _, rows, _ = emb.shape
    n_out = wbig.shape[2]
    blk = 512
    return pl.pallas_call(
        _mm_body,
        grid=(rows // blk,),
        in_specs=[
            pl.BlockSpec((N_TABLES, blk, 128), lambda i: (0, i, 0)),
            pl.BlockSpec(wbig.shape, lambda i: (0, 0, 0)),
            pl.BlockSpec((1, n_out), lambda i: (0, 0)),
        ],
        out_specs=pl.BlockSpec((blk, n_out), lambda i: (i, 0)),
        out_shape=jax.ShapeDtypeStruct((rows, n_out), jnp.float32),
    )(emb, wbig, b2d)


@jax.jit
def kernel(inputs, tables, W, b):
    n, vocab, dim = tables.shape
    batch = inputs.shape[1]
    packed = tables.reshape(n * vocab // 4, 128)
    offs = (jnp.arange(n, dtype=jnp.int32) * (vocab // 4))[:, None]
    pidx = (inputs >> 2) + offs
    sub_off = (inputs & 3) * dim
    pidx4 = pidx.reshape(n, NW, QS, SUB)
    sub4 = sub_off.reshape(n, NW, QS, SUB)
    emb_packed = _sc_gather(pidx4, sub4, packed)

    w3 = W.reshape(n, dim, 5)
    eye4 = jnp.eye(4, dtype=W.dtype)
    wbig = (eye4[None, :, None, :, None] * w3[:, None, :, None, :]).reshape(
        n, 128, 20
    )
    bias = jnp.tile(b, 4)[None, :]
    out_packed = _tc_matmul(emb_packed, wbig, bias)
    return out_packed.reshape(batch, 5)


# speedup vs baseline: 5.8298x; 5.8298x over previous
import functools

import jax
import jax.numpy as jnp
from jax import lax
from jax.experimental import pallas as pl
from jax.experimental.pallas import tpu as pltpu
from jax.experimental.pallas import tpu_sc as plsc

N_TABLES = 26
DIM = 32
N_OUT = 5
NC, NS = 2, 16
NW = NC * NS
VPAD = 100096
N_TASKS = N_TABLES * N_OUT
CHUNK = 2048


def _proj_body(w_ref, t_ref, out_ref):
    out_ref[0] = jax.lax.dot_general(
        w_ref[0],
        t_ref[0],
        (((1,), (0,)), ((), ())),
        preferred_element_type=jnp.float32,
    )


def _tc_project(w8, tables_t):
    n, dim, vocab = tables_t.shape
    blk = VPAD // 2
    return pl.pallas_call(
        _proj_body,
        grid=(n, 2),
        in_specs=[
            pl.BlockSpec((1, 8, dim), lambda j, c: (j, 0, 0)),
            pl.BlockSpec((1, dim, blk), lambda j, c: (j, 0, c)),
        ],
        out_specs=pl.BlockSpec((1, 8, blk), lambda j, c: (j, 0, c)),
        out_shape=jax.ShapeDtypeStruct((n, 8, VPAD), jnp.float32),
    )(w8, tables_t)


def _lookup_body(idx_hbm, p_hbm, val_hbm, row_v, idx_v, val_v, sem):
    wid = lax.axis_index("s") * NC + lax.axis_index("c")
    batch = idx_hbm.shape[1]
    n_chunks = batch // CHUNK

    @pl.loop(0, 5)
    def _task_loop(s):
        t = s * NW + wid

        @pl.when(t < N_TASKS)
        def _():
            j = t // N_OUT
            o = lax.rem(t, N_OUT)
            pltpu.sync_copy(p_hbm.at[j, o], row_v)

            @pl.loop(0, n_chunks)
            def _chunk(c):
                pltpu.sync_copy(idx_hbm.at[j, pl.ds(c * CHUNK, CHUNK)], idx_v)

                @pl.loop(0, CHUNK // 16)
                def _group(g):
                    iv = idx_v[pl.ds(g * 16, 16)]
                    val_v[pl.ds(g * 16, 16)] = plsc.load_gather(row_v, [iv])

                pltpu.sync_copy(
                    val_v, val_hbm.at[j, o, pl.ds(c * CHUNK, CHUNK)]
                )


def _sc_lookup(inputs, p):
    batch = inputs.shape[1]
    mesh = plsc.VectorSubcoreMesh(core_axis_name="c", subcore_axis_name="s")
    return pl.kernel(
        _lookup_body,
        out_type=jax.ShapeDtypeStruct((N_TABLES, 8, batch), jnp.float32),
        mesh=mesh,
        scratch_types=[
            pltpu.VMEM((VPAD,), jnp.float32),
            pltpu.VMEM((CHUNK,), jnp.int32),
            pltpu.VMEM((CHUNK,), jnp.float32),
            pltpu.SemaphoreType.DMA,
        ],
        compiler_params=pltpu.CompilerParams(
            use_tc_tiling_on_sc=False, needs_layout_passes=False
        ),
    )(inputs, p)


def _reduce_body(val_ref, b_ref, out_ref):
    acc = jnp.zeros(val_ref.shape[1:], dtype=jnp.float32)
    for j in range(N_TABLES):
        acc = acc + val_ref[j]
    out_ref[...] = acc[:N_OUT, :].T + b_ref[...]


def _tc_reduce(val, b2d):
    _, _, batch = val.shape
    blk = 4096
    return pl.pallas_call(
        _reduce_body,
        grid=(batch // blk,),
        in_specs=[
            pl.BlockSpec((N_TABLES, 8, blk), lambda i: (0, 0, i)),
            pl.BlockSpec((1, N_OUT), lambda i: (0, 0)),
        ],
        out_specs=pl.BlockSpec((blk, N_OUT), lambda i: (i, 0)),
        out_shape=jax.ShapeDtypeStruct((batch, N_OUT), jnp.float32),
    )(val, b2d)


@jax.jit
def kernel(inputs, tables, W, b):
    n, vocab, dim = tables.shape
    tables_t = jnp.transpose(tables, (0, 2, 1))
    w8 = jnp.zeros((n, 8, dim), W.dtype).at[:, :N_OUT, :].set(
        jnp.transpose(W.reshape(n, dim, N_OUT), (0, 2, 1))
    )
    p = _tc_project(w8, tables_t)
    val = _sc_lookup(inputs, p)
    return _tc_reduce(val, b.reshape(1, -1))
